# Initial kernel scaffold; baseline (speedup 1.0000x reference)
import jax
import jax.numpy as jnp
from jax.experimental import pallas as pl


def kernel(query, support, q_l1, q_dummy_l, q_deg_l, q_r1, q_dummy_r, q_deg_r, s_l1, s_dummy_l, s_deg_l, s_r1, s_dummy_r, s_deg_r, params):
    raise NotImplementedError("write your pallas kernel here")


# speedup vs baseline: 2.6499x; 2.6499x over previous
import jax
import jax.numpy as jnp
from jax import lax
from jax.experimental import pallas as pl
from jax.experimental.pallas import tpu as pltpu
from jax.experimental.pallas import tpu_sc as plsc

F32 = jnp.float32
I32 = jnp.int32

_B = 1024
_FEW = 5
_K = 50
_D = 64
_NQ = 2 * _B
_NS = 16
_GW = 16

_NW = 32
_CH = 128

_N1_RAW = 2 * _NQ * _K + 2 * _NS * _K + _NQ + _NS
_N1 = ((_N1_RAW + _NW * _CH - 1) // (_NW * _CH)) * (_NW * _CH)
_N2_RAW = _NQ * _K + _NS * _K
_N2 = ((_N2_RAW + _NW * _CH - 1) // (_NW * _CH)) * (_NW * _CH)
_C1 = _N1 // _NW // _CH
_C2 = _N2 // _NW // _CH


def _sc_gather_body(emb, idx, gtbl, gidx, out_e, out_g,
                    idx_v, rows_v, gidx_v, grows_v, sem):
    wid = lax.axis_index("s") * 2 + lax.axis_index("c")
    base1 = wid * (_N1 // _NW)

    def body1(i, carry):
        off = base1 + i * _CH
        pltpu.sync_copy(idx.at[pl.ds(off, _CH)], idx_v)
        pltpu.async_copy(emb.at[idx_v], rows_v, sem).wait()
        pltpu.sync_copy(rows_v, out_e.at[pl.ds(off, _CH)])
        return carry

    lax.fori_loop(0, _C1, body1, 0)

    base2 = wid * (_N2 // _NW)

    def body2(i, carry):
        off = base2 + i * _CH
        pltpu.sync_copy(gidx.at[pl.ds(off, _CH)], gidx_v)
        pltpu.async_copy(gtbl.at[gidx_v], grows_v, sem).wait()
        pltpu.sync_copy(grows_v, out_g.at[pl.ds(off, _CH)])
        return carry

    lax.fori_loop(0, _C2, body2, 0)


def _sc_gather(emb, idx, gtbl, gidx):
    mesh = plsc.VectorSubcoreMesh(core_axis_name="c", subcore_axis_name="s")
    fn = pl.kernel(
        _sc_gather_body,
        mesh=mesh,
        out_type=[
            jax.ShapeDtypeStruct((_N1, _D), F32),
            jax.ShapeDtypeStruct((_N2, _GW), F32),
        ],
        scratch_types=[
            pltpu.VMEM((_CH,), I32),
            pltpu.VMEM((_CH, _D), F32),
            pltpu.VMEM((_CH,), I32),
            pltpu.VMEM((_CH, _GW), F32),
            pltpu.SemaphoreType.DMA,
        ],
        compiler_params=pltpu.CompilerParams(use_tc_tiling_on_sc=False),
    )
    return fn(emb, idx, gtbl, gidx)


def _nbr_body(rel, ent, selfe, grow, gmod, degf, wt, wb, gb, temp,
              out, proj):
    bblk = out.shape[0]
    self_e = selfe[...]
    ns = jnp.sqrt(jnp.sum(self_e * self_e, axis=1, keepdims=True) + 1e-8)
    w = wt[...]
    bias = wb[...] + gb[...]
    lane16 = lax.broadcasted_iota(I32, (1, _GW), 1).astype(F32)
    cos_cols = []
    gsum = jnp.zeros((bblk, 1), F32)
    for j in range(_K):
        xj = jnp.concatenate([rel[:, j, :], ent[:, j, :]], axis=1)
        pj = jnp.dot(xj, w, preferred_element_type=F32) + bias
        pj = jnp.where(pj >= 0, pj, 0.01 * pj)
        proj[j] = pj
        num = jnp.sum(pj * self_e, axis=1, keepdims=True)
        nn = jnp.sqrt(jnp.sum(pj * pj, axis=1, keepdims=True) + 1e-8)
        cos_cols.append(num / (ns * nn + 1e-8))
        onehot = (lane16 == gmod[:, j:j + 1]).astype(F32)
        gsum = gsum + jnp.sum(grow[:, j, :] * onehot, axis=1, keepdims=True)
    cos = jnp.concatenate(cos_cols, axis=1)
    iota = lax.broadcasted_iota(I32, (bblk, _K), 1).astype(F32)
    selm = jnp.zeros((bblk, _K), F32)
    work = cos
    for _ in range(10):
        m = jnp.max(work, axis=1, keepdims=True)
        ism = work == m
        sel = jnp.min(jnp.where(ism, iota, 1e9), axis=1, keepdims=True)
        pick = iota == sel
        selm = selm + pick.astype(F32)
        work = jnp.where(pick, -1e30, work)
    wsel = selm * 0.1
    agg = jnp.zeros((bblk, _D), F32)
    for j in range(_K):
        agg = agg + wsel[:, j:j + 1] * proj[j]
    gate = jax.nn.sigmoid((gsum * (1.0 / _K)) / temp[0, 0])
    gate = jnp.where(degf[...] > 0, gate, 1.0)
    out[...] = jnp.tanh(self_e + gate * agg)


def _nbr_call(rel_e, ent_e, self_e, grow, gmod, degf, wt, wb, gb, temp, bblk):
    n = rel_e.shape[0]
    grid = n // bblk
    return pl.pallas_call(
        _nbr_body,
        grid=(grid,),
        in_specs=[
            pl.BlockSpec((bblk, _K, _D), lambda g: (g, 0, 0)),
            pl.BlockSpec((bblk, _K, _D), lambda g: (g, 0, 0)),
            pl.BlockSpec((bblk, _D), lambda g: (g, 0)),
            pl.BlockSpec((bblk, _K, _GW), lambda g: (g, 0, 0)),
            pl.BlockSpec((bblk, _K), lambda g: (g, 0)),
            pl.BlockSpec((bblk, 1), lambda g: (g, 0)),
            pl.BlockSpec((2 * _D, _D), lambda g: (0, 0)),
            pl.BlockSpec((1, _D), lambda g: (0, 0)),
            pl.BlockSpec((1, _D), lambda g: (0, 0)),
            pl.BlockSpec((1, 1), lambda g: (0, 0)),
        ],
        out_specs=pl.BlockSpec((bblk, _D), lambda g: (g, 0)),
        out_shape=jax.ShapeDtypeStruct((n, _D), F32),
        scratch_shapes=[pltpu.VMEM((_K, bblk, _D), F32)],
    )(rel_e, ent_e, self_e, grow, gmod, degf, wt, wb, gb, temp)


def _head_body(qv, sv, smask, w1, b1, w2, b2, gam, bet,
               wih, whh, bih, bhh, out):
    g_ = gam[...]
    bt = bet[...]

    def enc(x):
        h = jnp.maximum(jnp.dot(x, w1[...], preferred_element_type=F32) + b1[...], 0.0)
        o = jnp.dot(h, w2[...], preferred_element_type=F32) + b2[...]
        y = o + x
        mu = jnp.mean(y, axis=1, keepdims=True)
        d = y - mu
        var = jnp.mean(d * d, axis=1, keepdims=True)
        return g_ * d / jnp.sqrt(var + 1e-5) + bt

    sg = jnp.sum(enc(sv[...]) * smask[...], axis=0, keepdims=True) * (1.0 / _FEW)
    q = enc(qv[...])
    bblk = q.shape[0]
    qw = jnp.dot(q, wih[...], preferred_element_type=F32) + bih[...] + bhh[...]
    c = jnp.zeros((bblk, 256), F32)
    h = q
    hr = None
    for t in range(4):
        if t == 0:
            gates = qw
        else:
            gates = qw + jnp.dot(hr, whh[...], preferred_element_type=F32)
        ig = jax.nn.sigmoid(gates[:, 0:256])
        fg = jax.nn.sigmoid(gates[:, 256:512])
        gg = jnp.tanh(gates[:, 512:768])
        og = jax.nn.sigmoid(gates[:, 768:1024])
        c = fg * c + ig * gg
        hl = og * jnp.tanh(c)
        h = q + hl[:, 0:128]
        hr = jnp.concatenate([h, jnp.broadcast_to(sg, (bblk, 128))], axis=1)
    num = jnp.sum(h * sg, axis=1, keepdims=True)
    dq = jnp.sqrt(jnp.sum(h * h, axis=1, keepdims=True) + 1e-8)
    ds = jnp.sqrt(jnp.sum(sg * sg, axis=1, keepdims=True) + 1e-8)
    out[...] = num / (dq * ds)


def _head_call(qv, sv, smask, w1, b1, w2, b2, gam, bet, wih, whh, bih, bhh):
    bblk = 128
    grid = _B // bblk
    return pl.pallas_call(
        _head_body,
        grid=(grid,),
        in_specs=[
            pl.BlockSpec((bblk, 128), lambda g: (g, 0)),
            pl.BlockSpec((8, 128), lambda g: (0, 0)),
            pl.BlockSpec((8, 1), lambda g: (0, 0)),
            pl.BlockSpec((128, 256), lambda g: (0, 0)),
            pl.BlockSpec((1, 256), lambda g: (0, 0)),
            pl.BlockSpec((256, 128), lambda g: (0, 0)),
            pl.BlockSpec((1, 128), lambda g: (0, 0)),
            pl.BlockSpec((1, 128), lambda g: (0, 0)),
            pl.BlockSpec((1, 128), lambda g: (0, 0)),
            pl.BlockSpec((128, 1024), lambda g: (0, 0)),
            pl.BlockSpec((256, 1024), lambda g: (0, 0)),
            pl.BlockSpec((1, 1024), lambda g: (0, 0)),
            pl.BlockSpec((1, 1024), lambda g: (0, 0)),
        ],
        out_specs=pl.BlockSpec((bblk, 1), lambda g: (g, 0)),
        out_shape=jax.ShapeDtypeStruct((_B, 1), F32),
    )(qv, sv, smask, w1, b1, w2, b2, gam, bet, wih, whh, bih, bhh)


def kernel(query, support, q_l1, q_dummy_l, q_deg_l, q_r1, q_dummy_r,
           q_deg_r, s_l1, s_dummy_l, s_deg_l, s_r1, s_dummy_r, s_deg_r,
           params):
    p = params
    emb = p['symbol_emb']
    gtbl = p['gate_w'].reshape(62500, _GW)

    conn_q = jnp.concatenate([q_l1, q_r1], axis=0).astype(I32)
    pad3 = jnp.zeros((3, _K, 2), I32)
    conn_s = jnp.concatenate(
        [s_l1.astype(I32), pad3, s_r1.astype(I32), pad3], axis=0)
    rel_q = conn_q[:, :, 0]
    ent_q = conn_q[:, :, 1]
    rel_s = conn_s[:, :, 0]
    ent_s = conn_s[:, :, 1]
    z3 = jnp.zeros((3,), I32)
    self_q = jnp.concatenate([query[:, 0], query[:, 1]]).astype(I32)
    self_s = jnp.concatenate(
        [support[:, 0].astype(I32), z3, support[:, 1].astype(I32), z3])

    idx = jnp.concatenate([
        rel_q.ravel(), ent_q.ravel(), rel_s.ravel(), ent_s.ravel(),
        self_q, self_s, jnp.zeros((_N1 - _N1_RAW,), I32)])
    gidx = jnp.concatenate([
        rel_q.ravel() // _GW, rel_s.ravel() // _GW,
        jnp.zeros((_N2 - _N2_RAW,), I32)])

    E, G = _sc_gather(emb, idx, gtbl, gidx)

    nq = _NQ * _K
    ns = _NS * _K
    relq_e = E[0:nq].reshape(_NQ, _K, _D)
    entq_e = E[nq:2 * nq].reshape(_NQ, _K, _D)
    rels_e = E[2 * nq:2 * nq + ns].reshape(_NS, _K, _D)
    ents_e = E[2 * nq + ns:2 * nq + 2 * ns].reshape(_NS, _K, _D)
    o = 2 * nq + 2 * ns
    selfq_e = E[o:o + _NQ]
    selfs_e = E[o + _NQ:o + _NQ + _NS]
    growq = G[0:nq].reshape(_NQ, _K, _GW)
    grows = G[nq:nq + ns].reshape(_NS, _K, _GW)
    gmod_q = (rel_q % _GW).astype(F32)
    gmod_s = (rel_s % _GW).astype(F32)
    deg_q = jnp.concatenate([q_deg_l, q_deg_r]).astype(F32).reshape(_NQ, 1)
    zf3 = jnp.zeros((3,), F32)
    deg_s = jnp.concatenate(
        [s_deg_l.astype(F32), zf3, s_deg_r.astype(F32), zf3]).reshape(_NS, 1)

    wt = p['gcn_w_W'].T
    wb = p['gcn_w_b'].reshape(1, _D)
    gb = p['gcn_b'].reshape(1, _D)
    temp = p['gate_temp'].reshape(1, 1)

    q_out = _nbr_call(relq_e, entq_e, selfq_e, growq, gmod_q, deg_q,
                      wt, wb, gb, temp, 128)
    s_out = _nbr_call(rels_e, ents_e, selfs_e, grows, gmod_s, deg_s,
                      wt, wb, gb, temp, _NS)

    q_vec = jnp.concatenate([q_out[:_B], q_out[_B:]], axis=1)
    s_vec = jnp.concatenate([s_out[:8], s_out[8:]], axis=1)
    smask = (jnp.arange(8) < _FEW).astype(F32).reshape(8, 1)

    scores = _head_call(
        q_vec, s_vec, smask,
        p['se_W1'].T, p['se_b1'].reshape(1, 256),
        p['se_W2'].T, p['se_b2'].reshape(1, 128),
        p['se_gamma'].reshape(1, 128), p['se_beta'].reshape(1, 128),
        p['lstm_Wih'].T, p['lstm_Whh'].T,
        p['lstm_bih'].reshape(1, 1024), p['lstm_bhh'].reshape(1, 1024))
    return scores.reshape(_B)
